# baseline (speedup 1.0000x reference)
import functools

import jax
import jax.numpy as jnp
from jax import lax
from jax.experimental import pallas as pl
from jax.experimental.pallas import tpu as pltpu
from jax.experimental.pallas import tpu_sc as plsc

_NC = 2
_NS = 16
_NW = _NC * _NS

_CH = 128
_ROW_BLK = 1280

def _cdiv(a, b):
  return (a + b - 1) // b


def _make_sc_agg(n_pad, n_chunks, d):
  rows_per_tile = n_pad // _NS
  zero_chunks = rows_per_tile // 16
  out_chunks = rows_per_tile // _CH
  nk = d // 16
  mesh = plsc.VectorSubcoreMesh(core_axis_name="c", subcore_axis_name="s",
                                num_cores=_NC, num_subcores=_NS)

  @functools.partial(
      pl.kernel,
      out_type=jax.ShapeDtypeStruct((_NC, n_pad, d), jnp.float32),
      mesh=mesh,
      scratch_types=[
          pltpu.VMEM((4, _CH), jnp.int32),
          pltpu.VMEM((4, _CH), jnp.int32),
          pltpu.VMEM((_CH, d), jnp.float32),
          pltpu.VMEM((_CH, d), jnp.float32),
          pltpu.VMEM((16, d), jnp.float32),
          pltpu.VMEM_SHARED((n_pad, d), jnp.float32),
          pltpu.SemaphoreType.DMA,
          pltpu.SemaphoreType.DMA,
          pltpu.SemaphoreType.DMA,
          pltpu.SemaphoreType.DMA,
      ],
  )
  def sc_agg(h_hbm, meta_hbm, out_hbm,
             me_a, me_b, rows_a, rows_b, zbuf, agg_sh,
             sem_ga, sem_gb, sem_ma, sem_mb):
    cid = lax.axis_index("c")
    sid = lax.axis_index("s")
    wid = cid * _NS + sid
    last = n_chunks - 1

    zk = jnp.zeros((16,), jnp.float32)
    for r in range(16):
      for c in range(nk):
        zbuf[r, pl.ds(c * 16, 16)] = zk

    pltpu.async_copy(meta_hbm.at[wid, 0], me_a, sem_ma)
    pltpu.async_copy(meta_hbm.at[wid, 1], me_b, sem_mb)

    def zloop(r, carry):
      pltpu.sync_copy(zbuf, agg_sh.at[pl.ds((sid * zero_chunks + r) * 16, 16)])
      return carry

    lax.fori_loop(0, zero_chunks, zloop, 0)
    plsc.subcore_barrier()

    pltpu.make_async_copy(meta_hbm.at[wid, 0], me_a, sem_ma).wait()
    pltpu.async_copy(h_hbm.at[me_a.at[0]], rows_a, sem_ga)

    def compute(rows_v, me_v, accs):
      def gloop(g, accs):
        sf = jnp.astype(me_v[2, pl.ds(g * 16, 16)], jnp.float32)
        ef = jnp.astype(me_v[3, pl.ds(g * 16, 16)], jnp.float32)
        for ei in range(16):
          s = sf[ei]
          f = ef[ei]
          row = g * 16 + ei
          accs = tuple(
              accs[k] * s + rows_v[row, pl.ds(k * 16, 16)]
              for k in range(nk))
          for k in range(nk):
            rows_v[row, pl.ds(k * 16, 16)] = accs[k] * f
        return accs
      return lax.fori_loop(0, _CH // 16, gloop, accs)

    def half(j, accs, me_p, rows_p, sem_gp, sem_mp,
             me_q, rows_q, sem_gq, sem_mq):
      jn = jnp.minimum(j + 1, last)
      jn2 = jnp.minimum(j + 2, last)
      pltpu.make_async_copy(h_hbm.at[me_p.at[0]], rows_p, sem_gp).wait()
      pltpu.make_async_copy(meta_hbm.at[wid, jn], me_q, sem_mq).wait()
      pltpu.async_copy(h_hbm.at[me_q.at[0]], rows_q, sem_gq)
      accs = compute(rows_p, me_p, accs)
      pltpu.sync_copy(rows_p, agg_sh.at[me_p.at[1]], add=True)
      pltpu.async_copy(meta_hbm.at[wid, jn2], me_p, sem_mp)
      return accs

    def eloop(i, accs):
      j = i * 2
      accs = half(j, accs, me_a, rows_a, sem_ga, sem_ma,
                  me_b, rows_b, sem_gb, sem_mb)
      accs = half(j + 1, accs, me_b, rows_b, sem_gb, sem_mb,
                  me_a, rows_a, sem_ga, sem_ma)
      return accs

    zero_accs = tuple(jnp.zeros((16,), jnp.float32) for _ in range(nk))
    lax.fori_loop(0, n_chunks // 2, eloop, zero_accs)
    pltpu.make_async_copy(h_hbm.at[me_a.at[0]], rows_a, sem_ga).wait()
    pltpu.make_async_copy(meta_hbm.at[wid, 0], me_b, sem_mb).wait()
    plsc.subcore_barrier()

    def oloop(k, carry):
      base = sid * rows_per_tile + k * _CH
      pltpu.sync_copy(agg_sh.at[pl.ds(base, _CH)], rows_a)
      pltpu.sync_copy(rows_a, out_hbm.at[cid].at[pl.ds(base, _CH)])
      return carry

    lax.fori_loop(0, out_chunks, oloop, 0)

  return sc_agg


def _dot_t(x, w):
  return lax.dot_general(x, w, (((1,), (1,)), ((), ())),
                         preferred_element_type=jnp.float32)


def _dense0_body(x_ref, w_ref, b_ref, o_ref):
  o_ref[...] = jnp.tanh(_dot_t(x_ref[...], w_ref[...]) + b_ref[...])


def _layer_body(agg_ref, h_ref, wrel_ref, wroot_ref, b_ref, o_ref):
  a = agg_ref[0] + agg_ref[1]
  o_ref[...] = jnp.tanh((_dot_t(a, wrel_ref[...]) + b_ref[...]) +
                        _dot_t(h_ref[...], wroot_ref[...]))


def _final_body(h_ref, w_ref, b_ref, o_ref):
  o_ref[...] = jnp.maximum(
      _dot_t(h_ref[...], w_ref[...]) + b_ref[...], 0.0)


def _row_blocked(body, n_pad, d, in_specs):
  grid = (n_pad // _ROW_BLK,)
  return pl.pallas_call(
      body,
      grid=grid,
      in_specs=in_specs,
      out_specs=pl.BlockSpec((_ROW_BLK, d), lambda i: (i, 0)),
      out_shape=jax.ShapeDtypeStruct((n_pad, d), jnp.float32),
  )


def _mat_spec(d):
  return pl.BlockSpec((d, d), lambda i: (0, 0))


def _bias_spec(d):
  return pl.BlockSpec((1, d), lambda i: (0, 0))


def kernel(x, edge_index, W1, b1, Wrel, brel, Wroot, W2, b2):
  n, d = x.shape
  e = edge_index.shape[1]
  l = Wrel.shape[0]

  n_pad = _cdiv(n, _NS * _CH) * _NS * _CH
  n_chunks = 2 * _cdiv(_cdiv(e, _NW * _CH), 2)
  e_pad = _NW * n_chunks * _CH
  t_edges = n_chunks * _CH

  x_pad = jnp.pad(x, ((0, n_pad - n), (0, 0)))

  order = jnp.argsort(edge_index[1], stable=True)
  ds = jnp.concatenate(
      [edge_index[1][order], jnp.full((e_pad - e,), n, jnp.int32)])
  ss = jnp.concatenate(
      [edge_index[0][order], jnp.zeros((e_pad - e,), jnp.int32)])
  pos = jnp.arange(e_pad, dtype=jnp.int32)
  prev = jnp.concatenate([jnp.array([-1], jnp.int32), ds[:-1]])
  nxt = jnp.concatenate([ds[1:], jnp.array([-1], jnp.int32)])
  in_tile_pos = pos % t_edges
  samei = ((ds == prev) & (in_tile_pos != 0)).astype(jnp.int32)
  endi = ((ds != nxt) | (in_tile_pos == t_edges - 1)).astype(jnp.int32)

  meta4 = jnp.stack(
      [ss.reshape(_NW, n_chunks, _CH),
       ds.reshape(_NW, n_chunks, _CH),
       samei.reshape(_NW, n_chunks, _CH),
       endi.reshape(_NW, n_chunks, _CH)],
      axis=2)

  sc_agg = _make_sc_agg(n_pad, n_chunks, d)

  row_spec = pl.BlockSpec((_ROW_BLK, d), lambda i: (i, 0))
  agg_spec = pl.BlockSpec((_NC, _ROW_BLK, d), lambda i: (0, i, 0))

  dense0 = _row_blocked(_dense0_body, n_pad, d,
                        [row_spec, _mat_spec(d), _bias_spec(d)])
  layer = _row_blocked(_layer_body, n_pad, d,
                       [agg_spec, row_spec, _mat_spec(d), _mat_spec(d),
                        _bias_spec(d)])
  final = _row_blocked(_final_body, n_pad, d,
                       [row_spec, _mat_spec(d), _bias_spec(d)])

  h = dense0(x_pad, W1, b1.reshape(1, d))
  for i in range(l):
    agg = sc_agg(h, meta4)
    h = layer(agg, h, Wrel[i], Wroot[i], brel[i].reshape(1, d))
  out = final(h, W2, b2.reshape(1, d))
  return out[:n]


# speedup vs baseline: 1.0093x; 1.0093x over previous
import functools

import jax
import jax.numpy as jnp
from jax import lax
from jax.experimental import pallas as pl
from jax.experimental.pallas import tpu as pltpu
from jax.experimental.pallas import tpu_sc as plsc

_NC = 2
_NS = 16
_NW = _NC * _NS

_CH = 128
_ROW_BLK = 1280

def _cdiv(a, b):
  return (a + b - 1) // b


def _make_sc_agg(n_pad, n_chunks, d):
  rows_per_tile = n_pad // _NS
  zero_chunks = rows_per_tile // 16
  out_chunks = rows_per_tile // _CH
  nk = d // 16
  mesh = plsc.VectorSubcoreMesh(core_axis_name="c", subcore_axis_name="s",
                                num_cores=_NC, num_subcores=_NS)

  @functools.partial(
      pl.kernel,
      out_type=jax.ShapeDtypeStruct((_NC, n_pad, d), jnp.float32),
      mesh=mesh,
      scratch_types=[
          pltpu.VMEM((3, _CH), jnp.int32),
          pltpu.VMEM((3, _CH), jnp.int32),
          pltpu.VMEM((_CH, d), jnp.float32),
          pltpu.VMEM((_CH, d), jnp.float32),
          pltpu.VMEM((16, d), jnp.float32),
          pltpu.VMEM_SHARED((n_pad, d), jnp.float32),
          pltpu.SemaphoreType.DMA,
          pltpu.SemaphoreType.DMA,
          pltpu.SemaphoreType.DMA,
          pltpu.SemaphoreType.DMA,
      ],
  )
  def sc_agg(h_hbm, meta_hbm, out_hbm,
             me_a, me_b, rows_a, rows_b, zbuf, agg_sh,
             sem_ga, sem_gb, sem_ma, sem_mb):
    cid = lax.axis_index("c")
    sid = lax.axis_index("s")
    wid = cid * _NS + sid
    last = n_chunks - 1

    zk = jnp.zeros((16,), jnp.float32)
    for r in range(16):
      for c in range(nk):
        zbuf[r, pl.ds(c * 16, 16)] = zk

    pltpu.async_copy(meta_hbm.at[wid, 0], me_a, sem_ma)
    pltpu.async_copy(meta_hbm.at[wid, 1], me_b, sem_mb)

    def zloop(r, carry):
      pltpu.async_copy(
          zbuf, agg_sh.at[pl.ds((sid * zero_chunks + r) * 16, 16)], sem_ga)
      return carry

    lax.fori_loop(0, zero_chunks, zloop, 0)

    def zdrain(r, carry):
      pltpu.make_async_copy(
          zbuf, agg_sh.at[pl.ds((sid * zero_chunks + r) * 16, 16)],
          sem_ga).wait()
      return carry

    lax.fori_loop(0, zero_chunks, zdrain, 0)
    plsc.subcore_barrier()

    pltpu.make_async_copy(meta_hbm.at[wid, 0], me_a, sem_ma).wait()
    pltpu.async_copy(h_hbm.at[me_a.at[0]], rows_a, sem_ga)

    def compute(rows_v, me_v, accs):
      def gloop(g, accs):
        sf = jnp.astype(me_v[2, pl.ds(g * 16, 16)], jnp.float32)
        for ei in range(16):
          s = sf[ei]
          row = g * 16 + ei
          accs = tuple(
              accs[k] * s + rows_v[row, pl.ds(k * 16, 16)]
              for k in range(nk))
          for k in range(nk):
            rows_v[row, pl.ds(k * 16, 16)] = accs[k]
        return accs
      return lax.fori_loop(0, _CH // 16, gloop, accs)

    def half(j, accs, me_p, rows_p, sem_gp, sem_mp,
             me_q, rows_q, sem_gq, sem_mq):
      jn = jnp.minimum(j + 1, last)
      jn2 = jnp.minimum(j + 2, last)
      pltpu.make_async_copy(h_hbm.at[me_p.at[0]], rows_p, sem_gp).wait()
      pltpu.make_async_copy(meta_hbm.at[wid, jn], me_q, sem_mq).wait()
      pltpu.async_copy(h_hbm.at[me_q.at[0]], rows_q, sem_gq)
      accs = compute(rows_p, me_p, accs)
      pltpu.sync_copy(rows_p, agg_sh.at[me_p.at[1]], add=True)
      pltpu.async_copy(meta_hbm.at[wid, jn2], me_p, sem_mp)
      return accs

    def eloop(i, accs):
      j = i * 2
      accs = half(j, accs, me_a, rows_a, sem_ga, sem_ma,
                  me_b, rows_b, sem_gb, sem_mb)
      accs = half(j + 1, accs, me_b, rows_b, sem_gb, sem_mb,
                  me_a, rows_a, sem_ga, sem_ma)
      return accs

    zero_accs = tuple(jnp.zeros((16,), jnp.float32) for _ in range(nk))
    lax.fori_loop(0, n_chunks // 2, eloop, zero_accs)
    pltpu.make_async_copy(h_hbm.at[me_a.at[0]], rows_a, sem_ga).wait()
    pltpu.make_async_copy(meta_hbm.at[wid, 0], me_b, sem_mb).wait()
    plsc.subcore_barrier()

    def oloop(k, carry):
      base = sid * rows_per_tile + k * _CH
      pltpu.async_copy(agg_sh.at[pl.ds(base, _CH)],
                       out_hbm.at[cid].at[pl.ds(base, _CH)], sem_ga)
      return carry

    lax.fori_loop(0, out_chunks, oloop, 0)

    def odrain(k, carry):
      base = sid * rows_per_tile + k * _CH
      pltpu.make_async_copy(agg_sh.at[pl.ds(base, _CH)],
                            out_hbm.at[cid].at[pl.ds(base, _CH)],
                            sem_ga).wait()
      return carry

    lax.fori_loop(0, out_chunks, odrain, 0)

  return sc_agg


def _dot_t(x, w):
  return lax.dot_general(x, w, (((1,), (1,)), ((), ())),
                         preferred_element_type=jnp.float32)


def _dense0_body(x_ref, w_ref, b_ref, o_ref):
  o_ref[...] = jnp.tanh(_dot_t(x_ref[...], w_ref[...]) + b_ref[...])


def _layer_body(agg_ref, h_ref, wrel_ref, wroot_ref, b_ref, o_ref):
  a = agg_ref[0] + agg_ref[1]
  o_ref[...] = jnp.tanh((_dot_t(a, wrel_ref[...]) + b_ref[...]) +
                        _dot_t(h_ref[...], wroot_ref[...]))


def _final_body(h_ref, w_ref, b_ref, o_ref):
  o_ref[...] = jnp.maximum(
      _dot_t(h_ref[...], w_ref[...]) + b_ref[...], 0.0)


def _row_blocked(body, n_pad, d, in_specs):
  grid = (n_pad // _ROW_BLK,)
  return pl.pallas_call(
      body,
      grid=grid,
      in_specs=in_specs,
      out_specs=pl.BlockSpec((_ROW_BLK, d), lambda i: (i, 0)),
      out_shape=jax.ShapeDtypeStruct((n_pad, d), jnp.float32),
  )


def _mat_spec(d):
  return pl.BlockSpec((d, d), lambda i: (0, 0))


def _bias_spec(d):
  return pl.BlockSpec((1, d), lambda i: (0, 0))


def kernel(x, edge_index, W1, b1, Wrel, brel, Wroot, W2, b2):
  n, d = x.shape
  e = edge_index.shape[1]
  l = Wrel.shape[0]

  n_pad = _cdiv(n, _NS * _CH) * _NS * _CH
  if n_pad - n < 3 * _NW + 3:
    n_pad += _NS * _CH
  n_chunks = 2 * _cdiv(_cdiv(e, _NW * _CH), 2)
  e_pad = _NW * n_chunks * _CH
  t_edges = n_chunks * _CH

  x_pad = jnp.pad(x, ((0, n_pad - n), (0, 0)))

  order = jnp.argsort(edge_index[1], stable=True)
  ds = jnp.concatenate(
      [edge_index[1][order], jnp.full((e_pad - e,), n, jnp.int32)])
  ss = jnp.concatenate(
      [edge_index[0][order], jnp.zeros((e_pad - e,), jnp.int32)])
  pos = jnp.arange(e_pad, dtype=jnp.int32)
  prev = jnp.concatenate([jnp.array([-1], jnp.int32), ds[:-1]])
  nxt = jnp.concatenate([ds[1:], jnp.array([-1], jnp.int32)])
  in_tile_pos = pos % t_edges
  samei = ((ds == prev) & (in_tile_pos != 0)).astype(jnp.int32)
  is_end = (ds != nxt) | (in_tile_pos == t_edges - 1)
  sink = n + 1 + (pos // t_edges) * 3 + (pos % 3)
  sdst = jnp.where(is_end, ds, sink.astype(jnp.int32))

  meta4 = jnp.stack(
      [ss.reshape(_NW, n_chunks, _CH),
       sdst.reshape(_NW, n_chunks, _CH),
       samei.reshape(_NW, n_chunks, _CH)],
      axis=2)

  sc_agg = _make_sc_agg(n_pad, n_chunks, d)

  row_spec = pl.BlockSpec((_ROW_BLK, d), lambda i: (i, 0))
  agg_spec = pl.BlockSpec((_NC, _ROW_BLK, d), lambda i: (0, i, 0))

  dense0 = _row_blocked(_dense0_body, n_pad, d,
                        [row_spec, _mat_spec(d), _bias_spec(d)])
  layer = _row_blocked(_layer_body, n_pad, d,
                       [agg_spec, row_spec, _mat_spec(d), _mat_spec(d),
                        _bias_spec(d)])
  final = _row_blocked(_final_body, n_pad, d,
                       [row_spec, _mat_spec(d), _bias_spec(d)])

  h = dense0(x_pad, W1, b1.reshape(1, d))
  for i in range(l):
    agg = sc_agg(h, meta4)
    h = layer(agg, h, Wrel[i], Wroot[i], brel[i].reshape(1, d))
  out = final(h, W2, b2.reshape(1, d))
  return out[:n]
